# Initial kernel scaffold; baseline (speedup 1.0000x reference)
import functools

import jax
import jax.numpy as jnp
from jax import lax
from jax.experimental import pallas as pl
from jax.experimental.pallas import tpu as pltpu
from jax.experimental.pallas import tpu_sc as plsc

N = 100000
L = 200000
NW = 32
LANES = 16

LINK_CHUNK = 6272
LINK_PAD = NW * LINK_CHUNK
NODE_CHUNK = 3136
NODE_PAD = NW * NODE_CHUNK

_mesh = plsc.VectorSubcoreMesh(core_axis_name="c", subcore_axis_name="s")


def _wid():
    return lax.axis_index("s") * 2 + lax.axis_index("c")


def _grad_body(head_hbm, tail_hbm, len_hbm, array_hbm, grad_hbm,
               idxh_v, idxt_v, len_v, ah_v, at_v, grad_v, sem):
    base = _wid() * LINK_CHUNK
    pltpu.sync_copy(head_hbm.at[pl.ds(base, LINK_CHUNK)], idxh_v)
    pltpu.sync_copy(tail_hbm.at[pl.ds(base, LINK_CHUNK)], idxt_v)
    pltpu.sync_copy(len_hbm.at[pl.ds(base, LINK_CHUNK)], len_v)
    pltpu.async_copy(array_hbm.at[idxh_v], ah_v, sem).wait()
    pltpu.async_copy(array_hbm.at[idxt_v], at_v, sem).wait()

    def body(i, carry):
        sl = pl.ds(i * LANES, LANES)
        grad_v[sl] = (ah_v[sl] - at_v[sl]) / len_v[sl]
        return carry

    lax.fori_loop(0, LINK_CHUNK // LANES, body, 0)
    pltpu.sync_copy(grad_v, grad_hbm.at[pl.ds(base, LINK_CHUNK)])


_grad_kernel = functools.partial(
    pl.kernel,
    out_type=jax.ShapeDtypeStruct((LINK_PAD,), jnp.float32),
    mesh=_mesh,
    scratch_types=[
        pltpu.VMEM((LINK_CHUNK,), jnp.int32),
        pltpu.VMEM((LINK_CHUNK,), jnp.int32),
        pltpu.VMEM((LINK_CHUNK,), jnp.float32),
        pltpu.VMEM((LINK_CHUNK,), jnp.float32),
        pltpu.VMEM((LINK_CHUNK,), jnp.float32),
        pltpu.VMEM((LINK_CHUNK,), jnp.float32),
        pltpu.SemaphoreType.DMA,
    ],
)(_grad_body)


def _mean_body(linksT_hbm, grad_hbm, out_hbm, idx_v, g_v, out_v, sem):
    base = _wid() * NODE_CHUNK
    for j in range(4):
        pltpu.sync_copy(linksT_hbm.at[pl.ds(j * NODE_PAD + base, NODE_CHUNK)],
                        idx_v)
        pltpu.async_copy(grad_hbm.at[idx_v], g_v.at[j], sem).wait()

    def body(i, carry):
        sl = pl.ds(i * LANES, LANES)
        out_v[sl] = (g_v[0, sl] + g_v[1, sl] + g_v[2, sl] + g_v[3, sl]) * 0.25
        return carry

    lax.fori_loop(0, NODE_CHUNK // LANES, body, 0)
    pltpu.sync_copy(out_v, out_hbm.at[pl.ds(base, NODE_CHUNK)])


_mean_kernel = functools.partial(
    pl.kernel,
    out_type=jax.ShapeDtypeStruct((NODE_PAD,), jnp.float32),
    mesh=_mesh,
    scratch_types=[
        pltpu.VMEM((NODE_CHUNK,), jnp.int32),
        pltpu.VMEM((4, NODE_CHUNK), jnp.float32),
        pltpu.VMEM((NODE_CHUNK,), jnp.float32),
        pltpu.SemaphoreType.DMA,
    ],
)(_mean_kernel_body := _mean_body)


def kernel(array, length_of_link, node_at_link_head, node_at_link_tail,
           links_at_node):
    pad_l = LINK_PAD - L
    head_p = jnp.concatenate(
        [node_at_link_head, jnp.zeros((pad_l,), jnp.int32)])
    tail_p = jnp.concatenate(
        [node_at_link_tail, jnp.zeros((pad_l,), jnp.int32)])
    len_p = jnp.concatenate(
        [length_of_link, jnp.ones((pad_l,), jnp.float32)])
    linksT_p = jnp.concatenate(
        [links_at_node, jnp.zeros((NODE_PAD - N, 4), jnp.int32)]
    ).T.reshape(-1)

    grad = _grad_kernel(head_p, tail_p, len_p, array)
    out = _mean_kernel(linksT_p, grad)
    return out[:N]


# speedup vs baseline: 76.3954x; 76.3954x over previous
import functools

import jax
import jax.numpy as jnp
from jax import lax
from jax.experimental import pallas as pl
from jax.experimental.pallas import tpu as pltpu
from jax.experimental.pallas import tpu_sc as plsc

N = 100000
L = 200000
NW = 32
LANES = 16

LINK_CHUNK = 6272
LINK_PAD = NW * LINK_CHUNK
WORDS = LINK_PAD // 2
NODE_CHUNK = 3136
NODE_PAD = NW * NODE_CHUNK

_mesh = plsc.VectorSubcoreMesh(core_axis_name="c", subcore_axis_name="s")


def _wid():
    return lax.axis_index("s") * 2 + lax.axis_index("c")


def _bf16_hi(g):
    b = plsc.bitcast(g, jnp.int32)
    return (b + 0x8000) & jnp.int32(-65536)


def _grad_body(head_hbm, tail_hbm, len_hbm, array_hbm, w_hbm,
               arr_v, idxh_v, idxt_v, len_v, w_v, sem):
    base = _wid() * LINK_CHUNK
    pltpu.sync_copy(array_hbm, arr_v)
    pltpu.sync_copy(head_hbm.at[pl.ds(base, LINK_CHUNK)], idxh_v)
    pltpu.sync_copy(tail_hbm.at[pl.ds(base, LINK_CHUNK)], idxt_v)
    pltpu.sync_copy(len_hbm.at[pl.ds(base, LINK_CHUNK)], len_v)

    def body(m, carry):
        slu = pl.ds(m * 2 * LANES, LANES)
        slv = pl.ds(m * 2 * LANES + LANES, LANES)
        gu = (plsc.load_gather(arr_v, [idxh_v[slu]])
              - plsc.load_gather(arr_v, [idxt_v[slu]])) / len_v[slu]
        gv = (plsc.load_gather(arr_v, [idxh_v[slv]])
              - plsc.load_gather(arr_v, [idxt_v[slv]])) / len_v[slv]
        lo = lax.shift_right_logical(_bf16_hi(gu), 16)
        w_v[pl.ds(m * LANES, LANES)] = lo | _bf16_hi(gv)
        return carry

    lax.fori_loop(0, LINK_CHUNK // (2 * LANES), body, 0)
    pltpu.sync_copy(w_v, w_hbm.at[pl.ds(_wid() * (LINK_CHUNK // 2),
                                        LINK_CHUNK // 2)])


_grad_kernel = functools.partial(
    pl.kernel,
    out_type=jax.ShapeDtypeStruct((WORDS,), jnp.int32),
    mesh=_mesh,
    compiler_params=pltpu.CompilerParams(needs_layout_passes=False),
    scratch_types=[
        pltpu.VMEM((N,), jnp.float32),
        pltpu.VMEM((LINK_CHUNK,), jnp.int32),
        pltpu.VMEM((LINK_CHUNK,), jnp.int32),
        pltpu.VMEM((LINK_CHUNK,), jnp.float32),
        pltpu.VMEM((LINK_CHUNK // 2,), jnp.int32),
        pltpu.SemaphoreType.DMA,
    ],
)(_grad_body)


def _mean_body(linksT_hbm, w_hbm, out_hbm, w_v, idx_v, out_v, sem):
    base = _wid() * NODE_CHUNK
    pltpu.sync_copy(w_hbm, w_v)
    for j in range(4):
        pltpu.sync_copy(linksT_hbm.at[pl.ds(j * NODE_PAD + base, NODE_CHUNK)],
                        idx_v.at[pl.ds(j * NODE_CHUNK, NODE_CHUNK)])

    def body(i, carry):
        sl = pl.ds(i * LANES, LANES)
        acc = jnp.zeros((LANES,), jnp.float32)
        for j in range(4):
            l = idx_v[pl.ds(j * NODE_CHUNK + i * LANES, LANES)]
            k = lax.shift_left(lax.shift_right_logical(l, 5), 4) | (l & 15)
            w = plsc.load_gather(w_v, [k])
            bits = jnp.where((l & 16) != 0, w & jnp.int32(-65536),
                             lax.shift_left(w, 16))
            acc = acc + plsc.bitcast(bits, jnp.float32)
        out_v[sl] = acc * 0.25
        return carry

    lax.fori_loop(0, NODE_CHUNK // LANES, body, 0)
    pltpu.sync_copy(out_v, out_hbm.at[pl.ds(base, NODE_CHUNK)])


_mean_kernel = functools.partial(
    pl.kernel,
    out_type=jax.ShapeDtypeStruct((NODE_PAD,), jnp.float32),
    mesh=_mesh,
    compiler_params=pltpu.CompilerParams(needs_layout_passes=False),
    scratch_types=[
        pltpu.VMEM((WORDS,), jnp.int32),
        pltpu.VMEM((4 * NODE_CHUNK,), jnp.int32),
        pltpu.VMEM((NODE_CHUNK,), jnp.float32),
        pltpu.SemaphoreType.DMA,
    ],
)(_mean_body)


def kernel(array, length_of_link, node_at_link_head, node_at_link_tail,
           links_at_node):
    pad_l = LINK_PAD - L
    head_p = jnp.concatenate(
        [node_at_link_head, jnp.zeros((pad_l,), jnp.int32)])
    tail_p = jnp.concatenate(
        [node_at_link_tail, jnp.zeros((pad_l,), jnp.int32)])
    len_p = jnp.concatenate(
        [length_of_link, jnp.ones((pad_l,), jnp.float32)])
    linksT_p = jnp.concatenate(
        [links_at_node, jnp.zeros((NODE_PAD - N, 4), jnp.int32)]
    ).T.reshape(-1)

    packed = _grad_kernel(head_p, tail_p, len_p, array)
    out = _mean_kernel(linksT_p, packed)
    return out[:N]
